# Initial kernel scaffold; baseline (speedup 1.0000x reference)
import functools

import jax
import jax.numpy as jnp
from jax import lax
from jax.experimental import pallas as pl
from jax.experimental.pallas import tpu as pltpu
from jax.experimental.pallas import tpu_sc as plsc

N = 10000
E = 320000
IN_CH = 128
HID_CH = 256
OUT_CH = 128
BN_EPS = 1e-5

NC = 2
NS = 16
NW = NC * NS
EPW = E // NW
CHUNK = 125
NCHUNK = EPW // CHUNK
RPT = N // NS

AUG = 144


def _make_segsum(D):
    mesh = plsc.VectorSubcoreMesh(core_axis_name="c", subcore_axis_name="s")

    @functools.partial(
        pl.kernel,
        out_type=jax.ShapeDtypeStruct((NC, N, D), jnp.float32),
        mesh=mesh,
        scratch_types=[
            pltpu.VMEM((NCHUNK, CHUNK), jnp.int32),
            pltpu.VMEM((NCHUNK, CHUNK), jnp.int32),
            pltpu.VMEM((CHUNK, D), jnp.float32),
            pltpu.VMEM_SHARED((N, D), jnp.float32),
            pltpu.SemaphoreType.DMA,
        ],
    )
    def seg(x_hbm, src_hbm, dst_hbm, zeros_hbm, out_hbm,
            src_v, dst_v, rows, acc, sem):
        c = lax.axis_index("c")
        s = lax.axis_index("s")
        wid = s * NC + c
        pltpu.sync_copy(zeros_hbm.at[pl.ds(s * RPT, RPT)],
                        acc.at[pl.ds(s * RPT, RPT)])
        pltpu.sync_copy(src_hbm.at[wid], src_v)
        pltpu.sync_copy(dst_hbm.at[wid], dst_v)
        plsc.subcore_barrier()

        def body(j, carry):
            pltpu.async_copy(x_hbm.at[src_v.at[j]], rows, sem).wait()
            pltpu.sync_copy(rows, acc.at[dst_v.at[j]], add=True)
            return carry

        lax.fori_loop(0, NCHUNK, body, 0)
        plsc.subcore_barrier()
        pltpu.sync_copy(acc.at[pl.ds(s * RPT, RPT)],
                        out_hbm.at[c, pl.ds(s * RPT, RPT)])

    return seg


_seg_aug = _make_segsum(AUG)
_seg_out = _make_segsum(OUT_CH)


def _phase2(sums, x, W1_l, b1, W1_r, gamma, beta, W2_l, b2, W2_r):
    def body(sum_ref, x_ref, w1l_ref, b1_ref, w1r_ref, g_ref, be_ref,
             w2l_ref, b2_ref, w2r_ref, p_ref, r_ref, rinv_ref):
        tot = sum_ref[0] + sum_ref[1]
        cnt = tot[:, IN_CH:IN_CH + 1]
        rinv = 1.0 / jnp.maximum(cnt, 1.0)
        agg = tot[:, :IN_CH] * rinv
        h = (jnp.dot(agg, w1l_ref[...], preferred_element_type=jnp.float32)
             + b1_ref[...]
             + jnp.dot(x_ref[...], w1r_ref[...],
                       preferred_element_type=jnp.float32))
        mu = jnp.mean(h, axis=0, keepdims=True)
        var = jnp.mean((h - mu) ** 2, axis=0, keepdims=True)
        hn = (h - mu) / jnp.sqrt(var + BN_EPS) * g_ref[...] + be_ref[...]
        hr = jnp.maximum(hn, 0.0)
        p_ref[...] = jnp.dot(hr, w2l_ref[...],
                             preferred_element_type=jnp.float32)
        r_ref[...] = jnp.dot(hr, w2r_ref[...],
                             preferred_element_type=jnp.float32) + b2_ref[...]
        rinv_ref[...] = rinv

    return pl.pallas_call(
        body,
        out_shape=(
            jax.ShapeDtypeStruct((N, OUT_CH), jnp.float32),
            jax.ShapeDtypeStruct((N, OUT_CH), jnp.float32),
            jax.ShapeDtypeStruct((N, 1), jnp.float32),
        ),
    )(sums, x, W1_l, b1.reshape(1, HID_CH), W1_r,
      gamma.reshape(1, HID_CH), beta.reshape(1, HID_CH),
      W2_l, b2.reshape(1, OUT_CH), W2_r)


def _phase4(sums2, rinv, r):
    def body(sum_ref, rinv_ref, r_ref, out_ref):
        out_ref[...] = ((sum_ref[0] + sum_ref[1]) * rinv_ref[...]
                        + r_ref[...])

    return pl.pallas_call(
        body,
        out_shape=jax.ShapeDtypeStruct((N, OUT_CH), jnp.float32),
    )(sums2, rinv, r)


def kernel(x, edge_index, W1_l, b1, W1_r, gamma, beta, W2_l, b2, W2_r):
    src = edge_index[0].reshape(NW, NCHUNK, CHUNK)
    dst = edge_index[1].reshape(NW, NCHUNK, CHUNK)
    x_aug = jnp.concatenate(
        [x, jnp.ones((N, 1), x.dtype), jnp.zeros((N, AUG - IN_CH - 1),
                                                 x.dtype)], axis=1)
    z_aug = jnp.zeros((N, AUG), jnp.float32)
    z_out = jnp.zeros((N, OUT_CH), jnp.float32)

    sums1 = _seg_aug(x_aug, src, dst, z_aug)
    p, r, rinv = _phase2(sums1, x, W1_l, b1, W1_r, gamma, beta,
                         W2_l, b2, W2_r)
    sums2 = _seg_out(p, src, dst, z_out)
    return _phase4(sums2, rinv, r)


# speedup vs baseline: 9.0290x; 9.0290x over previous
import functools

import jax
import jax.numpy as jnp
from jax import lax
from jax.experimental import pallas as pl
from jax.experimental.pallas import tpu as pltpu
from jax.experimental.pallas import tpu_sc as plsc

N = 10000
E = 320000
IN_CH = 128
HID_CH = 256
OUT_CH = 128
BN_EPS = 1e-5

NC = 2
NS = 16
NW = NC * NS
EPW = E // NW
CHUNK = 125
NCHUNK = EPW // CHUNK
RPT = N // NS

AUG = 144


def _make_segsum(D):
    mesh = plsc.VectorSubcoreMesh(core_axis_name="c", subcore_axis_name="s")

    @functools.partial(
        pl.kernel,
        out_type=jax.ShapeDtypeStruct((NC, N, D), jnp.float32),
        mesh=mesh,
        compiler_params=pltpu.CompilerParams(use_tc_tiling_on_sc=False),
        scratch_types=[
            pltpu.VMEM((NCHUNK, CHUNK), jnp.int32),
            pltpu.VMEM((NCHUNK, CHUNK), jnp.int32),
            pltpu.VMEM((CHUNK, D), jnp.float32),
            pltpu.VMEM_SHARED((N, D), jnp.float32),
            pltpu.SemaphoreType.DMA,
        ],
    )
    def seg(x_hbm, src_hbm, dst_hbm, zeros_hbm, out_hbm,
            src_v, dst_v, rows, acc, sem):
        c = lax.axis_index("c")
        s = lax.axis_index("s")
        wid = s * NC + c
        pltpu.sync_copy(zeros_hbm.at[pl.ds(s * RPT, RPT)],
                        acc.at[pl.ds(s * RPT, RPT)])
        pltpu.sync_copy(src_hbm.at[wid], src_v)
        pltpu.sync_copy(dst_hbm.at[wid], dst_v)
        plsc.subcore_barrier()

        def body(j, carry):
            pltpu.async_copy(x_hbm.at[src_v.at[j]], rows, sem).wait()
            pltpu.sync_copy(rows, acc.at[dst_v.at[j]], add=True)
            return carry

        lax.fori_loop(0, NCHUNK, body, 0)
        plsc.subcore_barrier()
        pltpu.sync_copy(acc.at[pl.ds(s * RPT, RPT)],
                        out_hbm.at[c, pl.ds(s * RPT, RPT)])

    return seg


_seg_aug = _make_segsum(AUG)
_seg_out = _make_segsum(OUT_CH)


def _phase2(sums, x, W1_l, b1, W1_r, gamma, beta, W2_l, b2, W2_r):
    def body(sum_ref, x_ref, w1l_ref, b1_ref, w1r_ref, g_ref, be_ref,
             w2l_ref, b2_ref, w2r_ref, p_ref, r_ref, rinv_ref):
        tot = sum_ref[0] + sum_ref[1]
        cnt = tot[:, IN_CH:IN_CH + 1]
        rinv = 1.0 / jnp.maximum(cnt, 1.0)
        agg = tot[:, :IN_CH] * rinv
        h = (jnp.dot(agg, w1l_ref[...], preferred_element_type=jnp.float32)
             + b1_ref[...]
             + jnp.dot(x_ref[...], w1r_ref[...],
                       preferred_element_type=jnp.float32))
        mu = jnp.mean(h, axis=0, keepdims=True)
        var = jnp.mean((h - mu) ** 2, axis=0, keepdims=True)
        hn = (h - mu) / jnp.sqrt(var + BN_EPS) * g_ref[...] + be_ref[...]
        hr = jnp.maximum(hn, 0.0)
        p_ref[...] = jnp.dot(hr, w2l_ref[...],
                             preferred_element_type=jnp.float32)
        r_ref[...] = jnp.dot(hr, w2r_ref[...],
                             preferred_element_type=jnp.float32) + b2_ref[...]
        rinv_ref[...] = rinv

    return pl.pallas_call(
        body,
        out_shape=(
            jax.ShapeDtypeStruct((N, OUT_CH), jnp.float32),
            jax.ShapeDtypeStruct((N, OUT_CH), jnp.float32),
            jax.ShapeDtypeStruct((N, 1), jnp.float32),
        ),
    )(sums, x, W1_l, b1.reshape(1, HID_CH), W1_r,
      gamma.reshape(1, HID_CH), beta.reshape(1, HID_CH),
      W2_l, b2.reshape(1, OUT_CH), W2_r)


def _phase4(sums2, rinv, r):
    def body(sum_ref, rinv_ref, r_ref, out_ref):
        out_ref[...] = ((sum_ref[0] + sum_ref[1]) * rinv_ref[...]
                        + r_ref[...])

    return pl.pallas_call(
        body,
        out_shape=jax.ShapeDtypeStruct((N, OUT_CH), jnp.float32),
    )(sums2, rinv, r)


def kernel(x, edge_index, W1_l, b1, W1_r, gamma, beta, W2_l, b2, W2_r):
    src = edge_index[0].reshape(NW, NCHUNK, CHUNK)
    dst = edge_index[1].reshape(NW, NCHUNK, CHUNK)
    x_aug = jnp.concatenate(
        [x, jnp.ones((N, 1), x.dtype), jnp.zeros((N, AUG - IN_CH - 1),
                                                 x.dtype)], axis=1)
    z_aug = jnp.zeros((N, AUG), jnp.float32)
    z_out = jnp.zeros((N, OUT_CH), jnp.float32)

    sums1 = _seg_aug(x_aug, src, dst, z_aug)
    p, r, rinv = _phase2(sums1, x, W1_l, b1, W1_r, gamma, beta,
                         W2_l, b2, W2_r)
    sums2 = _seg_out(p, src, dst, z_out)
    return _phase4(sums2, rinv, r)


# speedup vs baseline: 12.3793x; 1.3711x over previous
import functools

import jax
import jax.numpy as jnp
from jax import lax
from jax.experimental import pallas as pl
from jax.experimental.pallas import tpu as pltpu
from jax.experimental.pallas import tpu_sc as plsc

N = 10000
E = 320000
IN_CH = 128
HID_CH = 256
OUT_CH = 128
BN_EPS = 1e-5

NC = 2
NS = 16
EPS_SC = E // NS
CHUNK = 125
NCHUNK = EPS_SC // CHUNK
RPT = N // NS

AUG = 160


def _make_segsum(DH):
    mesh = plsc.VectorSubcoreMesh(core_axis_name="c", subcore_axis_name="s")

    @functools.partial(
        pl.kernel,
        out_type=jax.ShapeDtypeStruct((N, 2 * DH), jnp.float32),
        mesh=mesh,
        compiler_params=pltpu.CompilerParams(use_tc_tiling_on_sc=False),
        scratch_types=[
            pltpu.VMEM((NCHUNK, CHUNK), jnp.int32),
            pltpu.VMEM((NCHUNK, CHUNK), jnp.int32),
            pltpu.VMEM((3, CHUNK, DH), jnp.float32),
            pltpu.VMEM_SHARED((N, DH), jnp.float32),
            pltpu.SemaphoreType.DMA,
            pltpu.SemaphoreType.DMA,
        ],
    )
    def seg(x2_hbm, src_hbm, dst_hbm, zeros_hbm, out_hbm,
            src_v, dst_v, rows3, acc, gsem, ssem):
        c = lax.axis_index("c")
        s = lax.axis_index("s")
        xc = x2_hbm.at[c]
        pltpu.sync_copy(zeros_hbm.at[pl.ds(s * RPT, RPT)],
                        acc.at[pl.ds(s * RPT, RPT)])
        pltpu.sync_copy(src_hbm.at[s], src_v)
        pltpu.sync_copy(dst_hbm.at[s], dst_v)
        plsc.subcore_barrier()

        def body(i, carry):
            @pl.when(i < NCHUNK)
            def _():
                pltpu.async_copy(xc.at[src_v.at[i]],
                                 rows3.at[lax.rem(i, 3)], gsem)

            @pl.when(jnp.logical_and(i >= 1, i <= NCHUNK))
            def _():
                pltpu.make_async_copy(zeros_hbm.at[pl.ds(0, CHUNK)],
                                      rows3.at[0], gsem).wait()
                j = i - 1
                pltpu.async_copy(rows3.at[lax.rem(j, 3)],
                                 acc.at[dst_v.at[j]], ssem, add=True)

            @pl.when(i >= 2)
            def _():
                pltpu.make_async_copy(zeros_hbm.at[pl.ds(0, CHUNK)],
                                      acc.at[pl.ds(0, CHUNK)], ssem).wait()

            return carry

        lax.fori_loop(0, NCHUNK + 2, body, 0)
        plsc.subcore_barrier()
        pltpu.sync_copy(acc.at[pl.ds(s * RPT, RPT)],
                        out_hbm.at[pl.ds(s * RPT, RPT), pl.ds(c * DH, DH)])

    return seg


_seg_aug = _make_segsum(AUG // 2)
_seg_out = _make_segsum(OUT_CH // 2)


def _phase2(sums, x, W1_l, b1, W1_r, gamma, beta, W2_l, b2, W2_r):
    def body(sum_ref, x_ref, w1l_ref, b1_ref, w1r_ref, g_ref, be_ref,
             w2l_ref, b2_ref, w2r_ref, p_ref, r_ref, rinv_ref):
        tot = sum_ref[...]
        cnt = tot[:, IN_CH:IN_CH + 1]
        rinv = 1.0 / jnp.maximum(cnt, 1.0)
        agg = tot[:, :IN_CH] * rinv
        h = (jnp.dot(agg, w1l_ref[...], preferred_element_type=jnp.float32)
             + b1_ref[...]
             + jnp.dot(x_ref[...], w1r_ref[...],
                       preferred_element_type=jnp.float32))
        mu = jnp.mean(h, axis=0, keepdims=True)
        var = jnp.mean((h - mu) ** 2, axis=0, keepdims=True)
        hn = (h - mu) / jnp.sqrt(var + BN_EPS) * g_ref[...] + be_ref[...]
        hr = jnp.maximum(hn, 0.0)
        p_ref[...] = jnp.dot(hr, w2l_ref[...],
                             preferred_element_type=jnp.float32)
        r_ref[...] = jnp.dot(hr, w2r_ref[...],
                             preferred_element_type=jnp.float32) + b2_ref[...]
        rinv_ref[...] = rinv

    return pl.pallas_call(
        body,
        out_shape=(
            jax.ShapeDtypeStruct((N, OUT_CH), jnp.float32),
            jax.ShapeDtypeStruct((N, OUT_CH), jnp.float32),
            jax.ShapeDtypeStruct((N, 1), jnp.float32),
        ),
    )(sums, x, W1_l, b1.reshape(1, HID_CH), W1_r,
      gamma.reshape(1, HID_CH), beta.reshape(1, HID_CH),
      W2_l, b2.reshape(1, OUT_CH), W2_r)


def _phase4(sums2, rinv, r):
    def body(sum_ref, rinv_ref, r_ref, out_ref):
        out_ref[...] = sum_ref[...] * rinv_ref[...] + r_ref[...]

    return pl.pallas_call(
        body,
        out_shape=jax.ShapeDtypeStruct((N, OUT_CH), jnp.float32),
    )(sums2, rinv, r)


def kernel(x, edge_index, W1_l, b1, W1_r, gamma, beta, W2_l, b2, W2_r):
    src = edge_index[0].reshape(NS, NCHUNK, CHUNK)
    dst = edge_index[1].reshape(NS, NCHUNK, CHUNK)
    x_aug = jnp.concatenate(
        [x, jnp.ones((N, 1), x.dtype),
         jnp.zeros((N, AUG - IN_CH - 1), x.dtype)], axis=1)
    DH1 = AUG // 2
    x2 = jnp.stack([x_aug[:, :DH1], x_aug[:, DH1:]])
    z1 = jnp.zeros((N, DH1), jnp.float32)
    DH2 = OUT_CH // 2
    z2 = jnp.zeros((N, DH2), jnp.float32)

    sums1 = _seg_aug(x2, src, dst, z1)
    p, r, rinv = _phase2(sums1, x, W1_l, b1, W1_r, gamma, beta,
                         W2_l, b2, W2_r)
    p2 = jnp.stack([p[:, :DH2], p[:, DH2:]])
    sums2 = _seg_out(p2, src, dst, z2)
    return _phase4(sums2, rinv, r)
